# baseline (speedup 1.0000x reference)
import functools

import jax
import jax.numpy as jnp
from jax import lax
from jax.experimental import pallas as pl
from jax.experimental.pallas import tpu as pltpu
from jax.experimental.pallas import tpu_sc as plsc

N_CB = 4
K = 1024
D = 64
ROWS = 32 * 576
NH = 2
HROWS = ROWS // NH
TILE = 1024
GRID = HROWS // TILE

_NC = 2
_NS = 16
_NW = _NC * _NS
BPW = HROWS // _NW
CH = 96
NCH = BPW // CH


def _dist_argmin(res, cb):
    c2 = jnp.sum(cb * cb, axis=1)
    dots = lax.dot_general(res, cb, (((1,), (1,)), ((), ())),
                           preferred_element_type=jnp.float32,
                           precision=lax.Precision.DEFAULT)
    r2 = jnp.sum(res * res, axis=1, keepdims=True)
    dist = (r2 - 2.0 * dots) + c2[None, :]
    return jnp.argmin(dist, axis=1).astype(jnp.int32)


def _tc_first_body(res_ref, cb_ref, idx_ref):
    idx_ref[...] = _dist_argmin(res_ref[...], cb_ref[...])


def _tc_step_body(res_ref, q_ref, cb_ref, idx_ref, newres_ref):
    res = res_ref[...] - q_ref[...]
    newres_ref[...] = res
    idx_ref[...] = _dist_argmin(res, cb_ref[...])


def _tc_final_body(x_ref, res_ref, q_ref, out_ref):
    out_ref[...] = x_ref[...] - res_ref[...] + q_ref[...]


def _tc_last_body(res_ref, q_ref, cb_ref, x_ref, idx_ref, part_ref):
    res = res_ref[...] - q_ref[...]
    part_ref[...] = x_ref[...] - res
    idx_ref[...] = _dist_argmin(res, cb_ref[...])


def _half_spec(off):
    return pl.BlockSpec((TILE, D), lambda i, off=off: (i + off, 0))


_row_spec = pl.BlockSpec((TILE, D), lambda i: (i, 0))
_cb_spec = pl.BlockSpec((K, D), lambda i: (0, 0))
_idx_spec = pl.BlockSpec((TILE,), lambda i: (i,))


def _tc_first(x, cb, h):
    return pl.pallas_call(
        _tc_first_body,
        grid=(GRID,),
        in_specs=[_half_spec(h * GRID), _cb_spec],
        out_specs=_idx_spec,
        out_shape=jax.ShapeDtypeStruct((HROWS,), jnp.int32),
    )(x, cb)


def _tc_step(res, res_spec, q, cb):
    return pl.pallas_call(
        _tc_step_body,
        grid=(GRID,),
        in_specs=[res_spec, _row_spec, _cb_spec],
        out_specs=[_idx_spec, _row_spec],
        out_shape=[jax.ShapeDtypeStruct((HROWS,), jnp.int32),
                   jax.ShapeDtypeStruct((HROWS, D), jnp.float32)],
    )(res, q, cb)


def _tc_final(x, res, q, h):
    return pl.pallas_call(
        _tc_final_body,
        grid=(GRID,),
        in_specs=[_half_spec(h * GRID), _row_spec, _row_spec],
        out_specs=_row_spec,
        out_shape=jax.ShapeDtypeStruct((HROWS, D), jnp.float32),
    )(x, res, q)


def _tc_last(res, q, cb, x, h):
    return pl.pallas_call(
        _tc_last_body,
        grid=(GRID,),
        in_specs=[_row_spec, _row_spec, _cb_spec, _half_spec(h * GRID)],
        out_specs=[_idx_spec, _row_spec],
        out_shape=[jax.ShapeDtypeStruct((HROWS,), jnp.int32),
                   jax.ShapeDtypeStruct((HROWS, D), jnp.float32)],
    )(res, q, cb, x)


def _sc_gather_body(cb_hbm, idx_hbm, out_hbm, idx_v, rows_v, isem, sem, wsem):
    wid = lax.axis_index("s") * _NC + lax.axis_index("c")
    base = wid * BPW
    pltpu.sync_copy(idx_hbm.at[wid], idx_v)
    gathers = []
    for j in range(NCH):
        gathers.append(pltpu.async_copy(
            cb_hbm.at[idx_v.at[j]], rows_v.at[pl.ds(j * CH, CH)], sem))
    writes = []
    for j in range(NCH):
        gathers[j].wait()
        writes.append(pltpu.async_copy(
            rows_v.at[pl.ds(j * CH, CH)],
            out_hbm.at[pl.ds(base + j * CH, CH)], wsem))
    for w in writes:
        w.wait()


@functools.lru_cache(maxsize=1)
def _sc_gather_call():
    return functools.partial(
        pl.kernel,
        mesh=plsc.VectorSubcoreMesh(core_axis_name="c", subcore_axis_name="s"),
        out_type=jax.ShapeDtypeStruct((HROWS, D), jnp.float32),
        scratch_types=[
            pltpu.VMEM((NCH, CH), jnp.int32),
            pltpu.VMEM((BPW, D), jnp.float32),
            pltpu.SemaphoreType.DMA,
            pltpu.SemaphoreType.DMA,
            pltpu.SemaphoreType.DMA,
        ],
        compiler_params=pltpu.CompilerParams(use_tc_tiling_on_sc=False),
    )(_sc_gather_body)


def _sc_gather(cb, idx):
    return _sc_gather_call()(cb, idx.reshape(_NW, NCH, CH))


def _sc_gather_add_body(cb_hbm, idx_hbm, part_hbm, out_hbm, idx_v, rows_v,
                        isem, psem, sem, wsem):
    wid = lax.axis_index("s") * _NC + lax.axis_index("c")
    base = wid * BPW
    pload = [pltpu.async_copy(part_hbm.at[pl.ds(base + j * CH, CH)],
                              rows_v.at[pl.ds(j * CH, CH)], psem)
             for j in range(NCH)]
    pltpu.sync_copy(idx_hbm.at[wid], idx_v)
    gathers = []
    for j in range(NCH):
        pload[j].wait()
        gathers.append(pltpu.async_copy(
            cb_hbm.at[idx_v.at[j]], rows_v.at[pl.ds(j * CH, CH)], sem,
            add=True))
    writes = []
    for j in range(NCH):
        gathers[j].wait()
        writes.append(pltpu.async_copy(
            rows_v.at[pl.ds(j * CH, CH)],
            out_hbm.at[pl.ds(base + j * CH, CH)], wsem))
    for w in writes:
        w.wait()


@functools.lru_cache(maxsize=1)
def _sc_gather_add_call():
    return functools.partial(
        pl.kernel,
        mesh=plsc.VectorSubcoreMesh(core_axis_name="c", subcore_axis_name="s"),
        out_type=jax.ShapeDtypeStruct((HROWS, D), jnp.float32),
        scratch_types=[
            pltpu.VMEM((NCH, CH), jnp.int32),
            pltpu.VMEM((BPW, D), jnp.float32),
            pltpu.SemaphoreType.DMA,
            pltpu.SemaphoreType.DMA,
            pltpu.SemaphoreType.DMA,
            pltpu.SemaphoreType.DMA,
        ],
        compiler_params=pltpu.CompilerParams(use_tc_tiling_on_sc=False),
    )(_sc_gather_add_body)


def _sc_gather_add(cb, idx, part):
    return _sc_gather_add_call()(cb, idx.reshape(_NW, NCH, CH), part)


def kernel(inputs, codebooks):
    x = inputs.reshape(ROWS, D)
    cbs = [codebooks[i] for i in range(N_CB)]

    idx = [None] * NH
    q = [None] * NH
    res = [None] * NH
    outs = [None] * NH

    for h in range(NH):
        idx[h] = _tc_first(x, cbs[0], h)
        q[h] = _sc_gather(cbs[0], idx[h])
    for i in range(1, N_CB - 1):
        for h in range(NH):
            prev, spec = ((x, _half_spec(h * GRID)) if i == 1
                          else (res[h], _row_spec))
            idx[h], res[h] = _tc_step(prev, spec, q[h], cbs[i])
            q[h] = _sc_gather(cbs[i], idx[h])
    for h in range(NH):
        idx[h], part = _tc_last(res[h], q[h], cbs[N_CB - 1], x, h)
        outs[h] = _sc_gather_add(cbs[N_CB - 1], idx[h], part)

    return jnp.concatenate(outs, axis=0).reshape(inputs.shape)


# speedup vs baseline: 1.0151x; 1.0151x over previous
import functools

import jax
import jax.numpy as jnp
from jax import lax
from jax.experimental import pallas as pl
from jax.experimental.pallas import tpu as pltpu
from jax.experimental.pallas import tpu_sc as plsc

N_CB = 4
K = 1024
D = 64
ROWS = 32 * 576
NH = 2
HROWS = ROWS // NH
TILE = 1024
GRID = HROWS // TILE

_NC = 2
_NS = 16
_NW = _NC * _NS
BPW = HROWS // _NW
CH = 96
NCH = BPW // CH


def _dist_argmin(res, cb):
    c2 = jnp.sum(cb * cb, axis=1)
    dots = lax.dot_general(res, cb, (((1,), (1,)), ((), ())),
                           preferred_element_type=jnp.float32,
                           precision=lax.Precision.DEFAULT)
    r2 = jnp.sum(res * res, axis=1, keepdims=True)
    dist = (r2 - 2.0 * dots) + c2[None, :]
    return jnp.argmin(dist, axis=1).astype(jnp.int32)


def _tc_first_body(res_ref, cb_ref, idx_ref):
    idx_ref[...] = _dist_argmin(res_ref[...], cb_ref[...])


def _tc_step_body(res_ref, q_ref, cb_ref, idx_ref, newres_ref):
    res = res_ref[...] - q_ref[...]
    newres_ref[...] = res
    idx_ref[...] = _dist_argmin(res, cb_ref[...])


def _tc_final_body(x_ref, res_ref, q_ref, out_ref):
    out_ref[...] = x_ref[...] - res_ref[...] + q_ref[...]


def _tc_last_body(res_ref, q_ref, cb_ref, x_ref, idx_ref, part_ref):
    res = res_ref[...] - q_ref[...]
    part_ref[...] = x_ref[...] - res
    idx_ref[...] = _dist_argmin(res, cb_ref[...])


def _half_spec(off):
    return pl.BlockSpec((TILE, D), lambda i, off=off: (i + off, 0))


_row_spec = pl.BlockSpec((TILE, D), lambda i: (i, 0))
_cb_spec = pl.BlockSpec((K, D), lambda i: (0, 0))
_idx_spec = pl.BlockSpec((TILE,), lambda i: (i,))


def _tc_first(x, cb, h):
    return pl.pallas_call(
        _tc_first_body,
        grid=(GRID,),
        in_specs=[_half_spec(h * GRID), _cb_spec],
        out_specs=_idx_spec,
        out_shape=jax.ShapeDtypeStruct((HROWS,), jnp.int32),
    )(x, cb)


def _tc_step(res, res_spec, q, cb):
    return pl.pallas_call(
        _tc_step_body,
        grid=(GRID,),
        in_specs=[res_spec, _row_spec, _cb_spec],
        out_specs=[_idx_spec, _row_spec],
        out_shape=[jax.ShapeDtypeStruct((HROWS,), jnp.int32),
                   jax.ShapeDtypeStruct((HROWS, D), jnp.float32)],
    )(res, q, cb)


def _tc_final(x, res, q, h):
    return pl.pallas_call(
        _tc_final_body,
        grid=(GRID,),
        in_specs=[_half_spec(h * GRID), _row_spec, _row_spec],
        out_specs=_row_spec,
        out_shape=jax.ShapeDtypeStruct((HROWS, D), jnp.float32),
    )(x, res, q)


def _tc_last(res, q, cb, x, h):
    return pl.pallas_call(
        _tc_last_body,
        grid=(GRID,),
        in_specs=[_row_spec, _row_spec, _cb_spec, _half_spec(h * GRID)],
        out_specs=[_idx_spec, _row_spec],
        out_shape=[jax.ShapeDtypeStruct((HROWS,), jnp.int32),
                   jax.ShapeDtypeStruct((HROWS, D), jnp.float32)],
    )(res, q, cb, x)


def _sc_gather_body(cb_hbm, idx_hbm, out_hbm, idx_v, rows_v, sem):
    wid = lax.axis_index("s") * _NC + lax.axis_index("c")
    base = wid * BPW
    pltpu.sync_copy(idx_hbm.at[wid], idx_v)
    copies = []
    for j in range(NCH):
        copies.append(pltpu.async_copy(
            cb_hbm.at[idx_v.at[j]], rows_v.at[pl.ds(j * CH, CH)], sem))
    for c in copies:
        c.wait()
    pltpu.sync_copy(rows_v, out_hbm.at[pl.ds(base, BPW)])


@functools.lru_cache(maxsize=1)
def _sc_gather_call():
    return functools.partial(
        pl.kernel,
        mesh=plsc.VectorSubcoreMesh(core_axis_name="c", subcore_axis_name="s"),
        out_type=jax.ShapeDtypeStruct((HROWS, D), jnp.float32),
        scratch_types=[
            pltpu.VMEM((NCH, CH), jnp.int32),
            pltpu.VMEM((BPW, D), jnp.float32),
            pltpu.SemaphoreType.DMA,
        ],
        compiler_params=pltpu.CompilerParams(use_tc_tiling_on_sc=False),
    )(_sc_gather_body)


def _sc_gather(cb, idx):
    return _sc_gather_call()(cb, idx.reshape(_NW, NCH, CH))


def _sc_gather_add_body(cb_hbm, idx_hbm, part_hbm, out_hbm, idx_v, rows_v, sem):
    wid = lax.axis_index("s") * _NC + lax.axis_index("c")
    base = wid * BPW
    pltpu.sync_copy(idx_hbm.at[wid], idx_v)
    pltpu.sync_copy(part_hbm.at[pl.ds(base, BPW)], rows_v)
    copies = []
    for j in range(NCH):
        copies.append(pltpu.async_copy(
            cb_hbm.at[idx_v.at[j]], rows_v.at[pl.ds(j * CH, CH)], sem,
            add=True))
    for c in copies:
        c.wait()
    pltpu.sync_copy(rows_v, out_hbm.at[pl.ds(base, BPW)])


@functools.lru_cache(maxsize=1)
def _sc_gather_add_call():
    return functools.partial(
        pl.kernel,
        mesh=plsc.VectorSubcoreMesh(core_axis_name="c", subcore_axis_name="s"),
        out_type=jax.ShapeDtypeStruct((HROWS, D), jnp.float32),
        scratch_types=[
            pltpu.VMEM((NCH, CH), jnp.int32),
            pltpu.VMEM((BPW, D), jnp.float32),
            pltpu.SemaphoreType.DMA,
        ],
        compiler_params=pltpu.CompilerParams(use_tc_tiling_on_sc=False),
    )(_sc_gather_add_body)


def _sc_gather_add(cb, idx, part):
    return _sc_gather_add_call()(cb, idx.reshape(_NW, NCH, CH), part)


def kernel(inputs, codebooks):
    x = inputs.reshape(ROWS, D)
    cbs = [codebooks[i] for i in range(N_CB)]

    idx = [None] * NH
    q = [None] * NH
    res = [None] * NH
    outs = [None] * NH

    for h in range(NH):
        idx[h] = _tc_first(x, cbs[0], h)
        q[h] = _sc_gather(cbs[0], idx[h])
    for i in range(1, N_CB - 1):
        for h in range(NH):
            prev, spec = ((x, _half_spec(h * GRID)) if i == 1
                          else (res[h], _row_spec))
            idx[h], res[h] = _tc_step(prev, spec, q[h], cbs[i])
            q[h] = _sc_gather(cbs[i], idx[h])
    for h in range(NH):
        idx[h], part = _tc_last(res[h], q[h], cbs[N_CB - 1], x, h)
        outs[h] = _sc_gather_add(cbs[N_CB - 1], idx[h], part)

    return jnp.concatenate(outs, axis=0).reshape(inputs.shape)
